# Initial kernel scaffold; baseline (speedup 1.0000x reference)
import jax
import jax.numpy as jnp
from jax.experimental import pallas as pl


def kernel(x, x_embed, y_embed):
    raise NotImplementedError("write your pallas kernel here")


# speedup vs baseline: 1.0025x; 1.0025x over previous
import jax
import jax.numpy as jnp
from jax.experimental import pallas as pl


def _body(x_ref, xe_ref, ye_ref, o_ref):
    ye = ye_ref[...]
    xe = xe_ref[...]
    pos = ye[:, None, :] + xe[None, :, :]
    o_ref[...] = x_ref[...] + pos[None, :, :, :]


def kernel(x, x_embed, y_embed):
    B, H, W, D = x.shape
    BB = 8
    grid = (B // BB,)
    return pl.pallas_call(
        _body,
        grid=grid,
        in_specs=[
            pl.BlockSpec((BB, H, W, D), lambda i: (i, 0, 0, 0)),
            pl.BlockSpec((W, D), lambda i: (0, 0)),
            pl.BlockSpec((H, D), lambda i: (0, 0)),
        ],
        out_specs=pl.BlockSpec((BB, H, W, D), lambda i: (i, 0, 0, 0)),
        out_shape=jax.ShapeDtypeStruct((B, H, W, D), x.dtype),
    )(x, x_embed, y_embed)


# speedup vs baseline: 1.0194x; 1.0168x over previous
import jax
import jax.numpy as jnp
from jax.experimental import pallas as pl


def _body(x_ref, xe_ref, ye_ref, o_ref):
    ye = ye_ref[...]
    xe = xe_ref[...]
    pos = ye[:, None, :] + xe[None, :, :]
    o_ref[...] = x_ref[...] + pos[None, :, :, :]


def kernel(x, x_embed, y_embed):
    B, H, W, D = x.shape
    BB = 16
    grid = (B // BB,)
    return pl.pallas_call(
        _body,
        grid=grid,
        in_specs=[
            pl.BlockSpec((BB, H, W, D), lambda i: (i, 0, 0, 0)),
            pl.BlockSpec((W, D), lambda i: (0, 0)),
            pl.BlockSpec((H, D), lambda i: (0, 0)),
        ],
        out_specs=pl.BlockSpec((BB, H, W, D), lambda i: (i, 0, 0, 0)),
        out_shape=jax.ShapeDtypeStruct((B, H, W, D), x.dtype),
    )(x, x_embed, y_embed)
